# baseline (speedup 1.0000x reference)
import functools

import jax
import jax.numpy as jnp
from jax import lax
from jax.experimental import pallas as pl
from jax.experimental.pallas import tpu as pltpu
from jax.experimental.pallas import tpu_sc as plsc

_K = 96
_NT = 16
_W = 128
_NPAD = 10240
_EPAD = 165888


def _make_agg(edge_split):
    n_workers = 2 * _NT if edge_split else _NT
    cpt = _EPAD // _K // n_workers
    rpt = _NPAD // _NT
    mesh = plsc.VectorSubcoreMesh(core_axis_name="c", subcore_axis_name="s")

    @functools.partial(
        pl.kernel,
        mesh=mesh,
        out_type=jax.ShapeDtypeStruct((2 * _NPAD, _W), jnp.float32),
        scratch_types=[
            pltpu.VMEM((6, 2, _K), jnp.int32),
            pltpu.VMEM((3, _K, _W), jnp.float32),
            pltpu.VMEM_SHARED((_NPAD, _W), jnp.float32),
            pltpu.SemaphoreType.DMA,
            pltpu.SemaphoreType.DMA,
            pltpu.SemaphoreType.DMA,
        ],
    )
    def agg_kernel(h_hbm, z_hbm, idx_hbm, out_hbm, idxb, rows, acc,
                   isem, gsem, ssem):
        c = lax.axis_index("c")
        s = lax.axis_index("s")
        rsl = pl.ds(s * rpt, rpt)
        pltpu.sync_copy(z_hbm.at[rsl, :], acc.at[rsl, :])
        plsc.subcore_barrier()

        cbase = (c * _NT + s) * cpt

        def ic(j, q):
            return pltpu.make_async_copy(idx_hbm.at[cbase + j], idxb.at[q],
                                         isem)

        def gat(q6, q3):
            return pltpu.make_async_copy(h_hbm.at[idxb.at[q6, 0]],
                                         rows.at[q3], gsem)

        def sca(q3, q6):
            return pltpu.make_async_copy(rows.at[q3], acc.at[idxb.at[q6, 1]],
                                         ssem)

        ic(0, 0).start()
        ic(1, 1).start()
        ic(2, 2).start()
        ic(0, 0).wait()
        ic(1, 1).wait()
        gat(0, 0).start()
        gat(1, 1).start()

        def body(i, carry):
            q6 = lax.rem(i, 6)
            q3 = lax.rem(i, 3)
            q6nn = lax.rem(i + 2, 6)
            q3nn = lax.rem(i + 2, 3)
            q6nnn = lax.rem(i + 3, 6)
            q3p = lax.rem(i + 2, 3)

            @pl.when(i >= 1)
            def _():
                sca(q3p, q6nn).wait()

            @pl.when(i + 2 < cpt)
            def _():
                ic(i + 2, q6nn).wait()

                @pl.when(i + 3 < cpt)
                def _():
                    ic(i + 3, q6nnn).start()

                gat(q6nn, q3nn).start()

            gat(q6, q3).wait()
            sca(q3, q6).start(add=True)
            return carry

        lax.fori_loop(0, cpt, body, 0)
        sca(0, 0).wait()
        plsc.subcore_barrier()
        pltpu.sync_copy(acc.at[rsl, :],
                        out_hbm.at[pl.ds(c * _NPAD + s * rpt, rpt), :])

    return agg_kernel


def _make_deg():
    cpt = _EPAD // _K // (2 * _NT)
    rpt = _NPAD // _NT
    mesh = plsc.VectorSubcoreMesh(core_axis_name="c", subcore_axis_name="s")

    @functools.partial(
        pl.kernel,
        mesh=mesh,
        out_type=jax.ShapeDtypeStruct((2 * _NPAD, _W), jnp.float32),
        scratch_types=[
            pltpu.VMEM((4, 2, _K), jnp.int32),
            pltpu.VMEM((_K, _W), jnp.float32),
            pltpu.VMEM_SHARED((_NPAD, _W), jnp.float32),
            pltpu.SemaphoreType.DMA,
            pltpu.SemaphoreType.DMA,
        ],
    )
    def deg_kernel(idx_hbm, ones_hbm, zero_hbm, out_hbm, idxb, ones, acc,
                   isem, ssem):
        c = lax.axis_index("c")
        s = lax.axis_index("s")
        pltpu.sync_copy(zero_hbm.at[pl.ds(s * rpt, rpt), :],
                        acc.at[pl.ds(s * rpt, rpt), :])
        pltpu.sync_copy(ones_hbm, ones)
        plsc.subcore_barrier()
        cbase = (c * _NT + s) * cpt

        def ic(j, q):
            return pltpu.make_async_copy(idx_hbm.at[cbase + j], idxb.at[q],
                                         isem)

        def sca(q):
            return pltpu.make_async_copy(ones, acc.at[idxb.at[q, 1]], ssem)

        ic(0, 0).start()
        ic(1, 1).start()

        def body(i, carry):
            q4 = lax.rem(i, 4)
            q4nn = lax.rem(i + 2, 4)
            ic(i, q4).wait()

            @pl.when(i >= 2)
            def _():
                sca(q4nn).wait()

            @pl.when(i + 2 < cpt)
            def _():
                ic(i + 2, q4nn).start()

            sca(q4).start(add=True)
            return carry

        lax.fori_loop(0, cpt, body, 0)
        sca(0).wait()
        sca(0).wait()
        plsc.subcore_barrier()
        pltpu.sync_copy(acc.at[pl.ds(s * rpt, rpt), :],
                        out_hbm.at[pl.ds(c * _NPAD + s * rpt, rpt), :])

    return deg_kernel


_RB = 512


def _tc_a_body(deg_ref, x_ref, w_ref, hp_ref, dinv_ref, *, n):
    r = pl.program_id(0)
    deg = deg_ref[0, :, 0:1] + deg_ref[1, :, 0:1] + 1.0
    dinv = lax.rsqrt(jnp.maximum(deg, 1.0))
    rows = r * _RB + lax.broadcasted_iota(jnp.int32, (_RB, 1), 0)
    dinv = jnp.where(rows < n, dinv, 0.0)
    h = jnp.dot(x_ref[...], w_ref[...], preferred_element_type=jnp.float32)
    hp = h * dinv
    hp_ref[0] = hp[:, :_W]
    hp_ref[1] = hp[:, _W:]
    dinv_ref[...] = dinv


def _tc_a(deg2, x_p, w):
    grid = _NPAD // _RB
    return pl.pallas_call(
        functools.partial(_tc_a_body, n=10000),
        grid=(grid,),
        in_specs=[
            pl.BlockSpec((2, _RB, _W), lambda r: (0, r, 0)),
            pl.BlockSpec((_RB, x_p.shape[1]), lambda r: (r, 0)),
            pl.BlockSpec(w.shape, lambda r: (0, 0)),
        ],
        out_specs=[
            pl.BlockSpec((2, _RB, _W), lambda r: (0, r, 0)),
            pl.BlockSpec((_RB, 1), lambda r: (r, 0)),
        ],
        out_shape=[
            jax.ShapeDtypeStruct((2, _NPAD, _W), jnp.float32),
            jax.ShapeDtypeStruct((_NPAD, 1), jnp.float32),
        ],
    )(deg2, x_p, w)


def _tc_b_body(agg_ref, hp_ref, dinv_ref, b_ref, g_ref, be_ref, w_ref,
               out_ref, colsum, colsq, *, n, split_out):
    p = pl.program_id(0)
    r = pl.program_id(1)
    a = jnp.concatenate([agg_ref[0], agg_ref[1]], axis=1)
    hsl = jnp.concatenate([hp_ref[0], hp_ref[1]], axis=1)
    t = (a + hsl) * dinv_ref[...] + b_ref[...]

    @pl.when((p == 0) & (r == 0))
    def _():
        colsum[...] = jnp.zeros_like(colsum)
        colsq[...] = jnp.zeros_like(colsq)

    @pl.when(p == 0)
    def _():
        rows = r * _RB + lax.broadcasted_iota(jnp.int32, (_RB, 1), 0)
        tm = jnp.where(rows < n, t, 0.0)
        colsum[...] += jnp.sum(tm, axis=0, keepdims=True)
        colsq[...] += jnp.sum(tm * tm, axis=0, keepdims=True)

    @pl.when(p == 1)
    def _():
        mean = colsum[...] / n
        var = colsq[...] / n - mean * mean
        y = (t - mean) * lax.rsqrt(var + 1e-5) * g_ref[...] + be_ref[...]
        y = jnp.maximum(y, 0.0)
        hnew = jnp.dot(y, w_ref[...], preferred_element_type=jnp.float32)
        hp = hnew * dinv_ref[...]
        if split_out:
            out_ref[0] = hp[:, :_W]
            out_ref[1] = hp[:, _W:]
        else:
            out_ref[0] = hp
            out_ref[1] = jnp.zeros_like(hp)


def _tc_b(agg, hp_in, dinv, b, g, be, w, split_out):
    hcur = 2 * _W
    grid = _NPAD // _RB
    return pl.pallas_call(
        functools.partial(_tc_b_body, n=10000, split_out=split_out),
        grid=(2, grid),
        in_specs=[
            pl.BlockSpec((2, _RB, _W), lambda p, r: (0, r, 0)),
            pl.BlockSpec((2, _RB, _W), lambda p, r: (0, r, 0)),
            pl.BlockSpec((_RB, 1), lambda p, r: (r, 0)),
            pl.BlockSpec((1, hcur), lambda p, r: (0, 0)),
            pl.BlockSpec((1, hcur), lambda p, r: (0, 0)),
            pl.BlockSpec((1, hcur), lambda p, r: (0, 0)),
            pl.BlockSpec(w.shape, lambda p, r: (0, 0)),
        ],
        out_specs=pl.BlockSpec((2, _RB, _W), lambda p, r: (0, r * p, 0)),
        out_shape=jax.ShapeDtypeStruct((2, _NPAD, _W), jnp.float32),
        scratch_shapes=[
            pltpu.VMEM((1, hcur), jnp.float32),
            pltpu.VMEM((1, hcur), jnp.float32),
        ],
    )(agg, hp_in, dinv, b, g, be, w)


def _tc_c_body(agg_ref, hp_ref, dinv_ref, b_ref, out_ref, *, c):
    t = ((agg_ref[0] + agg_ref[1] + hp_ref[0]) * dinv_ref[...]
         + b_ref[...])
    col = lax.broadcasted_iota(jnp.int32, t.shape, 1)
    tm = jnp.where(col < c, t, -jnp.inf)
    mx = jnp.max(tm, axis=1, keepdims=True)
    e = jnp.exp(tm - mx)
    lse = jnp.log(jnp.sum(e, axis=1, keepdims=True)) + mx
    out_ref[...] = t - lse


def _tc_c(agg, hp_in, dinv, b_p, c):
    grid = _NPAD // _RB
    return pl.pallas_call(
        functools.partial(_tc_c_body, c=c),
        grid=(grid,),
        in_specs=[
            pl.BlockSpec((2, _RB, _W), lambda r: (0, r, 0)),
            pl.BlockSpec((2, _RB, _W), lambda r: (0, r, 0)),
            pl.BlockSpec((_RB, 1), lambda r: (r, 0)),
            pl.BlockSpec((1, _W), lambda r: (0, 0)),
        ],
        out_specs=pl.BlockSpec((_RB, _W), lambda r: (r, 0)),
        out_shape=jax.ShapeDtypeStruct((_NPAD, _W), jnp.float32),
    )(agg, hp_in, dinv, b_p)


def kernel(x, edge_index, W0, b0, g0, be0, W1, b1, g1, be1, W2, b2):
    n, d = x.shape
    h = W0.shape[1]
    c = W2.shape[1]

    pad_e = _EPAD - edge_index.shape[1]
    src_p = jnp.concatenate(
        [edge_index[0], jnp.full((pad_e,), n, jnp.int32)])
    dst_p = jnp.concatenate(
        [edge_index[1], jnp.full((pad_e,), n, jnp.int32)])
    nch = _EPAD // _K
    idx_es = jnp.stack(
        [src_p.reshape(nch, _K), dst_p.reshape(nch, _K)], axis=1)
    idx_fs = jnp.concatenate(
        [idx_es, idx_es + jnp.array([_NPAD, 0], jnp.int32)[None, :, None]])
    x_p = jnp.pad(x, ((0, _NPAD - n), (0, 0)))
    w2_p = jnp.pad(W2, ((0, 0), (0, _W - c)))
    b2_p = jnp.pad(b2, ((0, _W - c),)).reshape(1, _W)
    z128 = jnp.zeros((_NPAD, _W), jnp.float32)

    agg_fs = _make_agg(edge_split=False)
    agg_es = _make_agg(edge_split=True)

    deg2 = _make_deg()(idx_es, jnp.ones((_K, _W), jnp.float32),
                       z128).reshape(2, _NPAD, _W)
    hp0, dinv = _tc_a(deg2, x_p, W0)

    a0 = agg_fs(hp0.reshape(2 * _NPAD, _W), z128, idx_fs).reshape(
        2, _NPAD, _W)
    hp1 = _tc_b(a0, hp0, dinv, b0.reshape(1, h), g0.reshape(1, h),
                be0.reshape(1, h), W1, split_out=True)
    a1 = agg_fs(hp1.reshape(2 * _NPAD, _W), z128, idx_fs).reshape(
        2, _NPAD, _W)
    hp2 = _tc_b(a1, hp1, dinv, b1.reshape(1, h), g1.reshape(1, h),
                be1.reshape(1, h), w2_p, split_out=False)

    a2 = agg_es(hp2.reshape(2 * _NPAD, _W), z128, idx_es).reshape(
        2, _NPAD, _W)
    out = _tc_c(a2, hp2, dinv, b2_p, c)
    return out[:n, :c]


# speedup vs baseline: 1.2087x; 1.2087x over previous
import functools

import jax
import jax.numpy as jnp
from jax import lax
from jax.experimental import pallas as pl
from jax.experimental.pallas import tpu as pltpu
from jax.experimental.pallas import tpu_sc as plsc

_K = 128
_NT = 16
_W = 128
_NPAD = 10240
_EPAD = 163840


def _make_agg(edge_split):
    n_workers = 2 * _NT if edge_split else _NT
    cpt = _EPAD // _K // n_workers
    rpt = _NPAD // _NT
    mesh = plsc.VectorSubcoreMesh(core_axis_name="c", subcore_axis_name="s")

    @functools.partial(
        pl.kernel,
        mesh=mesh,
        out_type=jax.ShapeDtypeStruct((2 * _NPAD, _W), jnp.float32),
        scratch_types=[
            pltpu.VMEM((4, 2, _K), jnp.int32),
            pltpu.VMEM((2, _K, _W), jnp.float32),
            pltpu.VMEM_SHARED((_NPAD, _W), jnp.float32),
            pltpu.SemaphoreType.DMA,
            pltpu.SemaphoreType.DMA,
            pltpu.SemaphoreType.DMA,
        ],
    )
    def agg_kernel(h_hbm, z_hbm, idx_hbm, out_hbm, idxb, rows, acc,
                   isem, gsem, ssem):
        c = lax.axis_index("c")
        s = lax.axis_index("s")
        rsl = pl.ds(s * rpt, rpt)
        pltpu.sync_copy(z_hbm.at[rsl, :], acc.at[rsl, :])
        plsc.subcore_barrier()

        cbase = (c * _NT + s) * cpt

        def ic(j, q):
            return pltpu.make_async_copy(idx_hbm.at[cbase + j], idxb.at[q],
                                         isem)

        def gat(q4, q2):
            return pltpu.make_async_copy(h_hbm.at[idxb.at[q4, 0]],
                                         rows.at[q2], gsem)

        def sca(q2, q4):
            return pltpu.make_async_copy(rows.at[q2], acc.at[idxb.at[q4, 1]],
                                         ssem)

        ic(0, 0).start()
        ic(0, 0).wait()
        ic(1, 1).start()
        gat(0, 0).start()

        def body(i, carry):
            q4 = lax.rem(i, 4)
            q2 = lax.rem(i, 2)
            q4n = lax.rem(i + 1, 4)
            q2n = lax.rem(i + 1, 2)
            q4nn = lax.rem(i + 2, 4)

            @pl.when(i + 1 < cpt)
            def _():
                ic(i + 1, q4n).wait()

                @pl.when(i >= 1)
                def _():
                    sca(q2n, q4n).wait()

                @pl.when(i + 2 < cpt)
                def _():
                    ic(i + 2, q4nn).start()

                gat(q4n, q2n).start()

            gat(q4, q2).wait()
            sca(q2, q4).start(add=True)
            return carry

        lax.fori_loop(0, cpt, body, 0)
        sca(0, 0).wait()
        sca(0, 0).wait()
        plsc.subcore_barrier()
        pltpu.sync_copy(acc.at[rsl, :],
                        out_hbm.at[pl.ds(c * _NPAD + s * rpt, rpt), :])

    return agg_kernel


def _make_deg():
    cpt = _EPAD // _K // (2 * _NT)
    rpt = _NPAD // _NT
    mesh = plsc.VectorSubcoreMesh(core_axis_name="c", subcore_axis_name="s")

    @functools.partial(
        pl.kernel,
        mesh=mesh,
        out_type=jax.ShapeDtypeStruct((2 * _NPAD, _W), jnp.float32),
        scratch_types=[
            pltpu.VMEM((4, 2, _K), jnp.int32),
            pltpu.VMEM((_K, _W), jnp.float32),
            pltpu.VMEM_SHARED((_NPAD, _W), jnp.float32),
            pltpu.SemaphoreType.DMA,
            pltpu.SemaphoreType.DMA,
        ],
    )
    def deg_kernel(idx_hbm, ones_hbm, zero_hbm, out_hbm, idxb, ones, acc,
                   isem, ssem):
        c = lax.axis_index("c")
        s = lax.axis_index("s")
        pltpu.sync_copy(zero_hbm.at[pl.ds(s * rpt, rpt), :],
                        acc.at[pl.ds(s * rpt, rpt), :])
        pltpu.sync_copy(ones_hbm, ones)
        plsc.subcore_barrier()
        cbase = (c * _NT + s) * cpt

        def ic(j, q):
            return pltpu.make_async_copy(idx_hbm.at[cbase + j], idxb.at[q],
                                         isem)

        def sca(q):
            return pltpu.make_async_copy(ones, acc.at[idxb.at[q, 1]], ssem)

        ic(0, 0).start()
        ic(1, 1).start()

        def body(i, carry):
            q4 = lax.rem(i, 4)
            q4nn = lax.rem(i + 2, 4)
            ic(i, q4).wait()

            @pl.when(i >= 2)
            def _():
                sca(q4nn).wait()

            @pl.when(i + 2 < cpt)
            def _():
                ic(i + 2, q4nn).start()

            sca(q4).start(add=True)
            return carry

        lax.fori_loop(0, cpt, body, 0)
        sca(0).wait()
        sca(0).wait()
        plsc.subcore_barrier()
        pltpu.sync_copy(acc.at[pl.ds(s * rpt, rpt), :],
                        out_hbm.at[pl.ds(c * _NPAD + s * rpt, rpt), :])

    return deg_kernel


_RB = 512


def _tc_a_body(deg_ref, x_ref, w_ref, hp_ref, dinv_ref, *, n):
    r = pl.program_id(0)
    deg = deg_ref[0, :, 0:1] + deg_ref[1, :, 0:1] + 1.0
    dinv = lax.rsqrt(jnp.maximum(deg, 1.0))
    rows = r * _RB + lax.broadcasted_iota(jnp.int32, (_RB, 1), 0)
    dinv = jnp.where(rows < n, dinv, 0.0)
    h = jnp.dot(x_ref[...], w_ref[...], preferred_element_type=jnp.float32)
    hp = h * dinv
    hp_ref[0] = hp[:, :_W]
    hp_ref[1] = hp[:, _W:]
    dinv_ref[...] = dinv


def _tc_a(deg2, x_p, w):
    grid = _NPAD // _RB
    return pl.pallas_call(
        functools.partial(_tc_a_body, n=10000),
        grid=(grid,),
        in_specs=[
            pl.BlockSpec((2, _RB, _W), lambda r: (0, r, 0)),
            pl.BlockSpec((_RB, x_p.shape[1]), lambda r: (r, 0)),
            pl.BlockSpec(w.shape, lambda r: (0, 0)),
        ],
        out_specs=[
            pl.BlockSpec((2, _RB, _W), lambda r: (0, r, 0)),
            pl.BlockSpec((_RB, 1), lambda r: (r, 0)),
        ],
        out_shape=[
            jax.ShapeDtypeStruct((2, _NPAD, _W), jnp.float32),
            jax.ShapeDtypeStruct((_NPAD, 1), jnp.float32),
        ],
    )(deg2, x_p, w)


def _tc_b_body(agg_ref, hp_ref, dinv_ref, b_ref, g_ref, be_ref, w_ref,
               out_ref, colsum, colsq, *, n, split_out):
    p = pl.program_id(0)
    r = pl.program_id(1)
    a = jnp.concatenate([agg_ref[0], agg_ref[1]], axis=1)
    hsl = jnp.concatenate([hp_ref[0], hp_ref[1]], axis=1)
    t = (a + hsl) * dinv_ref[...] + b_ref[...]

    @pl.when((p == 0) & (r == 0))
    def _():
        colsum[...] = jnp.zeros_like(colsum)
        colsq[...] = jnp.zeros_like(colsq)

    @pl.when(p == 0)
    def _():
        rows = r * _RB + lax.broadcasted_iota(jnp.int32, (_RB, 1), 0)
        tm = jnp.where(rows < n, t, 0.0)
        colsum[...] += jnp.sum(tm, axis=0, keepdims=True)
        colsq[...] += jnp.sum(tm * tm, axis=0, keepdims=True)

    @pl.when(p == 1)
    def _():
        mean = colsum[...] / n
        var = colsq[...] / n - mean * mean
        y = (t - mean) * lax.rsqrt(var + 1e-5) * g_ref[...] + be_ref[...]
        y = jnp.maximum(y, 0.0)
        hnew = jnp.dot(y, w_ref[...], preferred_element_type=jnp.float32)
        hp = hnew * dinv_ref[...]
        if split_out:
            out_ref[0] = hp[:, :_W]
            out_ref[1] = hp[:, _W:]
        else:
            out_ref[0] = hp
            out_ref[1] = jnp.zeros_like(hp)


def _tc_b(agg, hp_in, dinv, b, g, be, w, split_out):
    hcur = 2 * _W
    grid = _NPAD // _RB
    return pl.pallas_call(
        functools.partial(_tc_b_body, n=10000, split_out=split_out),
        grid=(2, grid),
        in_specs=[
            pl.BlockSpec((2, _RB, _W), lambda p, r: (0, r, 0)),
            pl.BlockSpec((2, _RB, _W), lambda p, r: (0, r, 0)),
            pl.BlockSpec((_RB, 1), lambda p, r: (r, 0)),
            pl.BlockSpec((1, hcur), lambda p, r: (0, 0)),
            pl.BlockSpec((1, hcur), lambda p, r: (0, 0)),
            pl.BlockSpec((1, hcur), lambda p, r: (0, 0)),
            pl.BlockSpec(w.shape, lambda p, r: (0, 0)),
        ],
        out_specs=pl.BlockSpec((2, _RB, _W), lambda p, r: (0, r * p, 0)),
        out_shape=jax.ShapeDtypeStruct((2, _NPAD, _W), jnp.float32),
        scratch_shapes=[
            pltpu.VMEM((1, hcur), jnp.float32),
            pltpu.VMEM((1, hcur), jnp.float32),
        ],
    )(agg, hp_in, dinv, b, g, be, w)


def _tc_c_body(agg_ref, hp_ref, dinv_ref, b_ref, out_ref, *, c):
    t = ((agg_ref[0] + agg_ref[1] + hp_ref[0]) * dinv_ref[...]
         + b_ref[...])
    col = lax.broadcasted_iota(jnp.int32, t.shape, 1)
    tm = jnp.where(col < c, t, -jnp.inf)
    mx = jnp.max(tm, axis=1, keepdims=True)
    e = jnp.exp(tm - mx)
    lse = jnp.log(jnp.sum(e, axis=1, keepdims=True)) + mx
    out_ref[...] = t - lse


def _tc_c(agg, hp_in, dinv, b_p, c):
    grid = _NPAD // _RB
    return pl.pallas_call(
        functools.partial(_tc_c_body, c=c),
        grid=(grid,),
        in_specs=[
            pl.BlockSpec((2, _RB, _W), lambda r: (0, r, 0)),
            pl.BlockSpec((2, _RB, _W), lambda r: (0, r, 0)),
            pl.BlockSpec((_RB, 1), lambda r: (r, 0)),
            pl.BlockSpec((1, _W), lambda r: (0, 0)),
        ],
        out_specs=pl.BlockSpec((_RB, _W), lambda r: (r, 0)),
        out_shape=jax.ShapeDtypeStruct((_NPAD, _W), jnp.float32),
    )(agg, hp_in, dinv, b_p)


def kernel(x, edge_index, W0, b0, g0, be0, W1, b1, g1, be1, W2, b2):
    n, d = x.shape
    h = W0.shape[1]
    c = W2.shape[1]

    pad_e = _EPAD - edge_index.shape[1]
    src_p = jnp.concatenate(
        [edge_index[0], jnp.full((pad_e,), n, jnp.int32)])
    dst_p = jnp.concatenate(
        [edge_index[1], jnp.full((pad_e,), n, jnp.int32)])
    nch = _EPAD // _K
    idx_es = jnp.stack(
        [src_p.reshape(nch, _K), dst_p.reshape(nch, _K)], axis=1)
    idx_fs = jnp.concatenate(
        [idx_es, idx_es + jnp.array([_NPAD, 0], jnp.int32)[None, :, None]])
    x_p = jnp.pad(x, ((0, _NPAD - n), (0, 0)))
    w2_p = jnp.pad(W2, ((0, 0), (0, _W - c)))
    b2_p = jnp.pad(b2, ((0, _W - c),)).reshape(1, _W)
    z128 = jnp.zeros((_NPAD, _W), jnp.float32)

    agg_fs = _make_agg(edge_split=False)
    agg_es = _make_agg(edge_split=True)

    deg2 = _make_deg()(idx_es, jnp.ones((_K, _W), jnp.float32),
                       z128).reshape(2, _NPAD, _W)
    hp0, dinv = _tc_a(deg2, x_p, W0)

    a0 = agg_fs(hp0.reshape(2 * _NPAD, _W), z128, idx_fs).reshape(
        2, _NPAD, _W)
    hp1 = _tc_b(a0, hp0, dinv, b0.reshape(1, h), g0.reshape(1, h),
                be0.reshape(1, h), W1, split_out=True)
    a1 = agg_fs(hp1.reshape(2 * _NPAD, _W), z128, idx_fs).reshape(
        2, _NPAD, _W)
    hp2 = _tc_b(a1, hp1, dinv, b1.reshape(1, h), g1.reshape(1, h),
                be1.reshape(1, h), w2_p, split_out=False)

    a2 = agg_es(hp2.reshape(2 * _NPAD, _W), z128, idx_es).reshape(
        2, _NPAD, _W)
    out = _tc_c(a2, hp2, dinv, b2_p, c)
    return out[:n, :c]


# speedup vs baseline: 1.2526x; 1.0364x over previous
import functools

import jax
import jax.numpy as jnp
from jax import lax
from jax.experimental import pallas as pl
from jax.experimental.pallas import tpu as pltpu
from jax.experimental.pallas import tpu_sc as plsc

_K = 128
_NT = 16
_W = 128
_NPAD = 10240
_EPAD = 163840


def _make_agg(edge_split):
    n_workers = 2 * _NT if edge_split else _NT
    cpt = _EPAD // _K // n_workers
    rpt = _NPAD // _NT
    mesh = plsc.VectorSubcoreMesh(core_axis_name="c", subcore_axis_name="s")

    @functools.partial(
        pl.kernel,
        mesh=mesh,
        out_type=jax.ShapeDtypeStruct((2 * _NPAD, _W), jnp.float32),
        scratch_types=[
            pltpu.VMEM((4, 2, _K), jnp.int32),
            pltpu.VMEM((2, _K, _W), jnp.float32),
            pltpu.VMEM_SHARED((_NPAD, _W), jnp.float32),
            pltpu.SemaphoreType.DMA,
            pltpu.SemaphoreType.DMA,
            pltpu.SemaphoreType.DMA,
        ],
    )
    def agg_kernel(h_hbm, z_hbm, idx_hbm, out_hbm, idxb, rows, acc,
                   isem, gsem, ssem):
        c = lax.axis_index("c")
        s = lax.axis_index("s")
        rsl = pl.ds(s * rpt, rpt)
        pltpu.sync_copy(z_hbm.at[rsl, :], acc.at[rsl, :])
        plsc.subcore_barrier()

        cbase = (c * _NT + s) * cpt

        def ic(j, q):
            return pltpu.make_async_copy(idx_hbm.at[cbase + j], idxb.at[q],
                                         isem)

        def gat(q4, q2):
            return pltpu.make_async_copy(h_hbm.at[idxb.at[q4, 0]],
                                         rows.at[q2], gsem)

        def sca(q2, q4):
            return pltpu.make_async_copy(rows.at[q2], acc.at[idxb.at[q4, 1]],
                                         ssem)

        ic(0, 0).start()
        ic(0, 0).wait()
        ic(1, 1).start()
        gat(0, 0).start()

        def body(i, carry):
            q4 = lax.rem(i, 4)
            q2 = lax.rem(i, 2)
            q4n = lax.rem(i + 1, 4)
            q2n = lax.rem(i + 1, 2)
            q4nn = lax.rem(i + 2, 4)

            @pl.when(i + 1 < cpt)
            def _():
                ic(i + 1, q4n).wait()

                @pl.when(i >= 1)
                def _():
                    sca(q2n, q4n).wait()

                @pl.when(i + 2 < cpt)
                def _():
                    ic(i + 2, q4nn).start()

                gat(q4n, q2n).start()

            gat(q4, q2).wait()
            sca(q2, q4).start(add=True)
            return carry

        lax.fori_loop(0, cpt, body, 0)
        sca(0, 0).wait()
        sca(0, 0).wait()
        plsc.subcore_barrier()
        pltpu.sync_copy(acc.at[rsl, :],
                        out_hbm.at[pl.ds(c * _NPAD + s * rpt, rpt), :])

    return agg_kernel


def _make_deg():
    cpt = _EPAD // _K // (2 * _NT)
    rpt = _NPAD // _NT
    mesh = plsc.VectorSubcoreMesh(core_axis_name="c", subcore_axis_name="s")

    @functools.partial(
        pl.kernel,
        mesh=mesh,
        out_type=jax.ShapeDtypeStruct((2 * _NPAD, _W), jnp.float32),
        scratch_types=[
            pltpu.VMEM((4, 2, _K), jnp.int32),
            pltpu.VMEM((_K, _W), jnp.float32),
            pltpu.VMEM_SHARED((_NPAD, _W), jnp.float32),
            pltpu.SemaphoreType.DMA,
            pltpu.SemaphoreType.DMA,
        ],
    )
    def deg_kernel(idx_hbm, ones_hbm, zero_hbm, out_hbm, idxb, ones, acc,
                   isem, ssem):
        c = lax.axis_index("c")
        s = lax.axis_index("s")
        pltpu.sync_copy(zero_hbm.at[pl.ds(s * rpt, rpt), :],
                        acc.at[pl.ds(s * rpt, rpt), :])
        pltpu.sync_copy(ones_hbm, ones)
        plsc.subcore_barrier()
        cbase = (c * _NT + s) * cpt

        def ic(j, q):
            return pltpu.make_async_copy(idx_hbm.at[cbase + j], idxb.at[q],
                                         isem)

        def sca(q):
            return pltpu.make_async_copy(ones, acc.at[idxb.at[q, 1]], ssem)

        ic(0, 0).start()
        ic(1, 1).start()

        def body(i, carry):
            q4 = lax.rem(i, 4)
            q4nn = lax.rem(i + 2, 4)
            ic(i, q4).wait()

            @pl.when(i >= 2)
            def _():
                sca(q4nn).wait()

            @pl.when(i + 2 < cpt)
            def _():
                ic(i + 2, q4nn).start()

            sca(q4).start(add=True)
            return carry

        lax.fori_loop(0, cpt, body, 0)
        sca(0).wait()
        sca(0).wait()
        plsc.subcore_barrier()
        pltpu.sync_copy(acc.at[pl.ds(s * rpt, rpt), :],
                        out_hbm.at[pl.ds(c * _NPAD + s * rpt, rpt), :])

    return deg_kernel


_RB = 1024


def _tc_a_body(deg_ref, x_ref, w_ref, hp_ref, dinv_ref, *, n):
    r = pl.program_id(0)
    deg = deg_ref[0, :, 0:1] + deg_ref[1, :, 0:1] + 1.0
    dinv = lax.rsqrt(jnp.maximum(deg, 1.0))
    rows = r * _RB + lax.broadcasted_iota(jnp.int32, (_RB, 1), 0)
    dinv = jnp.where(rows < n, dinv, 0.0)
    h = jnp.dot(x_ref[...], w_ref[...], preferred_element_type=jnp.float32)
    hp = h * dinv
    hp_ref[0] = hp[:, :_W]
    hp_ref[1] = hp[:, _W:]
    dinv_ref[...] = dinv


def _tc_a(deg2, x_p, w):
    grid = _NPAD // _RB
    return pl.pallas_call(
        functools.partial(_tc_a_body, n=10000),
        grid=(grid,),
        in_specs=[
            pl.BlockSpec((2, _RB, _W), lambda r: (0, r, 0)),
            pl.BlockSpec((_RB, x_p.shape[1]), lambda r: (r, 0)),
            pl.BlockSpec(w.shape, lambda r: (0, 0)),
        ],
        out_specs=[
            pl.BlockSpec((2, _RB, _W), lambda r: (0, r, 0)),
            pl.BlockSpec((_RB, 1), lambda r: (r, 0)),
        ],
        out_shape=[
            jax.ShapeDtypeStruct((2, _NPAD, _W), jnp.float32),
            jax.ShapeDtypeStruct((_NPAD, 1), jnp.float32),
        ],
    )(deg2, x_p, w)


def _tc_b_body(agg_ref, hp_ref, dinv_ref, b_ref, g_ref, be_ref, w_ref,
               out_ref, colsum, colsq, *, n, split_out):
    p = pl.program_id(0)
    r = pl.program_id(1)
    a = jnp.concatenate([agg_ref[0], agg_ref[1]], axis=1)
    hsl = jnp.concatenate([hp_ref[0], hp_ref[1]], axis=1)
    t = (a + hsl) * dinv_ref[...] + b_ref[...]

    @pl.when((p == 0) & (r == 0))
    def _():
        colsum[...] = jnp.zeros_like(colsum)
        colsq[...] = jnp.zeros_like(colsq)

    @pl.when(p == 0)
    def _():
        rows = r * _RB + lax.broadcasted_iota(jnp.int32, (_RB, 1), 0)
        tm = jnp.where(rows < n, t, 0.0)
        colsum[...] += jnp.sum(tm, axis=0, keepdims=True)
        colsq[...] += jnp.sum(tm * tm, axis=0, keepdims=True)

    @pl.when(p == 1)
    def _():
        mean = colsum[...] / n
        var = colsq[...] / n - mean * mean
        y = (t - mean) * lax.rsqrt(var + 1e-5) * g_ref[...] + be_ref[...]
        y = jnp.maximum(y, 0.0)
        hnew = jnp.dot(y, w_ref[...], preferred_element_type=jnp.float32)
        hp = hnew * dinv_ref[...]
        if split_out:
            out_ref[0] = hp[:, :_W]
            out_ref[1] = hp[:, _W:]
        else:
            out_ref[0] = hp
            out_ref[1] = jnp.zeros_like(hp)


def _tc_b(agg, hp_in, dinv, b, g, be, w, split_out):
    hcur = 2 * _W
    grid = _NPAD // _RB
    return pl.pallas_call(
        functools.partial(_tc_b_body, n=10000, split_out=split_out),
        grid=(2, grid),
        in_specs=[
            pl.BlockSpec((2, _RB, _W), lambda p, r: (0, r, 0)),
            pl.BlockSpec((2, _RB, _W), lambda p, r: (0, r, 0)),
            pl.BlockSpec((_RB, 1), lambda p, r: (r, 0)),
            pl.BlockSpec((1, hcur), lambda p, r: (0, 0)),
            pl.BlockSpec((1, hcur), lambda p, r: (0, 0)),
            pl.BlockSpec((1, hcur), lambda p, r: (0, 0)),
            pl.BlockSpec(w.shape, lambda p, r: (0, 0)),
        ],
        out_specs=pl.BlockSpec((2, _RB, _W), lambda p, r: (0, r * p, 0)),
        out_shape=jax.ShapeDtypeStruct((2, _NPAD, _W), jnp.float32),
        scratch_shapes=[
            pltpu.VMEM((1, hcur), jnp.float32),
            pltpu.VMEM((1, hcur), jnp.float32),
        ],
    )(agg, hp_in, dinv, b, g, be, w)


def _tc_c_body(agg_ref, hp_ref, dinv_ref, b_ref, out_ref, *, c):
    t = ((agg_ref[0] + agg_ref[1] + hp_ref[0]) * dinv_ref[...]
         + b_ref[...])
    col = lax.broadcasted_iota(jnp.int32, t.shape, 1)
    tm = jnp.where(col < c, t, -jnp.inf)
    mx = jnp.max(tm, axis=1, keepdims=True)
    e = jnp.exp(tm - mx)
    lse = jnp.log(jnp.sum(e, axis=1, keepdims=True)) + mx
    out_ref[...] = t - lse


def _tc_c(agg, hp_in, dinv, b_p, c):
    grid = _NPAD // _RB
    return pl.pallas_call(
        functools.partial(_tc_c_body, c=c),
        grid=(grid,),
        in_specs=[
            pl.BlockSpec((2, _RB, _W), lambda r: (0, r, 0)),
            pl.BlockSpec((2, _RB, _W), lambda r: (0, r, 0)),
            pl.BlockSpec((_RB, 1), lambda r: (r, 0)),
            pl.BlockSpec((1, _W), lambda r: (0, 0)),
        ],
        out_specs=pl.BlockSpec((_RB, _W), lambda r: (r, 0)),
        out_shape=jax.ShapeDtypeStruct((_NPAD, _W), jnp.float32),
    )(agg, hp_in, dinv, b_p)


def kernel(x, edge_index, W0, b0, g0, be0, W1, b1, g1, be1, W2, b2):
    n, d = x.shape
    h = W0.shape[1]
    c = W2.shape[1]

    pad_e = _EPAD - edge_index.shape[1]
    src_p = jnp.concatenate(
        [edge_index[0], jnp.full((pad_e,), n, jnp.int32)])
    dst_p = jnp.concatenate(
        [edge_index[1], jnp.full((pad_e,), n, jnp.int32)])
    nch = _EPAD // _K
    idx_es = jnp.stack(
        [src_p.reshape(nch, _K), dst_p.reshape(nch, _K)], axis=1)
    idx_fs = jnp.concatenate(
        [idx_es, idx_es + jnp.array([_NPAD, 0], jnp.int32)[None, :, None]])
    x_p = jnp.pad(x, ((0, _NPAD - n), (0, 0)))
    w2_p = jnp.pad(W2, ((0, 0), (0, _W - c)))
    b2_p = jnp.pad(b2, ((0, _W - c),)).reshape(1, _W)
    z128 = jnp.zeros((_NPAD, _W), jnp.float32)

    agg_fs = _make_agg(edge_split=False)
    agg_es = _make_agg(edge_split=True)

    deg2 = _make_deg()(idx_es, jnp.ones((_K, _W), jnp.float32),
                       z128).reshape(2, _NPAD, _W)
    hp0, dinv = _tc_a(deg2, x_p, W0)

    a0 = agg_fs(hp0.reshape(2 * _NPAD, _W), z128, idx_fs).reshape(
        2, _NPAD, _W)
    hp1 = _tc_b(a0, hp0, dinv, b0.reshape(1, h), g0.reshape(1, h),
                be0.reshape(1, h), W1, split_out=True)
    a1 = agg_fs(hp1.reshape(2 * _NPAD, _W), z128, idx_fs).reshape(
        2, _NPAD, _W)
    hp2 = _tc_b(a1, hp1, dinv, b1.reshape(1, h), g1.reshape(1, h),
                be1.reshape(1, h), w2_p, split_out=False)

    a2 = agg_es(hp2.reshape(2 * _NPAD, _W), z128, idx_es).reshape(
        2, _NPAD, _W)
    out = _tc_c(a2, hp2, dinv, b2_p, c)
    return out[:n, :c]


# speedup vs baseline: 1.2698x; 1.0137x over previous
import functools

import jax
import jax.numpy as jnp
from jax import lax
from jax.experimental import pallas as pl
from jax.experimental.pallas import tpu as pltpu
from jax.experimental.pallas import tpu_sc as plsc

_K = 128
_NT = 16
_W = 128
_NPAD = 10240
_EPAD = 163840


def _make_agg(edge_split):
    n_workers = 2 * _NT if edge_split else _NT
    cpt = _EPAD // _K // n_workers
    rpt = _NPAD // _NT
    mesh = plsc.VectorSubcoreMesh(core_axis_name="c", subcore_axis_name="s")

    @functools.partial(
        pl.kernel,
        mesh=mesh,
        out_type=jax.ShapeDtypeStruct((2 * _NPAD, _W), jnp.float32),
        scratch_types=[
            pltpu.VMEM((4, 2, _K), jnp.int32),
            pltpu.VMEM((2, _K, _W), jnp.float32),
            pltpu.VMEM_SHARED((_NPAD, _W), jnp.float32),
            pltpu.SemaphoreType.DMA,
            pltpu.SemaphoreType.DMA,
            pltpu.SemaphoreType.DMA,
        ],
    )
    def agg_kernel(h_hbm, z_hbm, idx_hbm, out_hbm, idxb, rows, acc,
                   isem, gsem, ssem):
        c = lax.axis_index("c")
        s = lax.axis_index("s")
        rsl = pl.ds(s * rpt, rpt)
        pltpu.sync_copy(z_hbm.at[rsl, :], acc.at[rsl, :])
        plsc.subcore_barrier()

        cbase = (c * _NT + s) * cpt

        def ic(j, q):
            return pltpu.make_async_copy(idx_hbm.at[cbase + j], idxb.at[q],
                                         isem)

        def gat(q4, q2):
            return pltpu.make_async_copy(h_hbm.at[idxb.at[q4, 0]],
                                         rows.at[q2], gsem)

        def sca(q2, q4):
            return pltpu.make_async_copy(rows.at[q2], acc.at[idxb.at[q4, 1]],
                                         ssem)

        ic(0, 0).start()
        ic(0, 0).wait()
        ic(1, 1).start()
        gat(0, 0).start()

        def body(i, carry):
            q4 = lax.rem(i, 4)
            q2 = lax.rem(i, 2)
            q4n = lax.rem(i + 1, 4)
            q2n = lax.rem(i + 1, 2)
            q4nn = lax.rem(i + 2, 4)

            @pl.when(i + 1 < cpt)
            def _():
                ic(i + 1, q4n).wait()

                @pl.when(i >= 1)
                def _():
                    sca(q2n, q4n).wait()

                @pl.when(i + 2 < cpt)
                def _():
                    ic(i + 2, q4nn).start()

                gat(q4n, q2n).start()

            gat(q4, q2).wait()
            sca(q2, q4).start(add=True)
            return carry

        lax.fori_loop(0, cpt, body, 0)
        sca(0, 0).wait()
        sca(0, 0).wait()
        plsc.subcore_barrier()
        pltpu.sync_copy(acc.at[rsl, :],
                        out_hbm.at[pl.ds(c * _NPAD + s * rpt, rpt), :])

    return agg_kernel


def _make_deg():
    cpt = _EPAD // _K // (2 * _NT)
    rpt = _NPAD // _NT
    mesh = plsc.VectorSubcoreMesh(core_axis_name="c", subcore_axis_name="s")

    @functools.partial(
        pl.kernel,
        mesh=mesh,
        out_type=jax.ShapeDtypeStruct((2 * _NPAD, _W), jnp.float32),
        scratch_types=[
            pltpu.VMEM((4, 2, _K), jnp.int32),
            pltpu.VMEM((_K, _W), jnp.float32),
            pltpu.VMEM_SHARED((_NPAD, _W), jnp.float32),
            pltpu.SemaphoreType.DMA,
            pltpu.SemaphoreType.DMA,
        ],
    )
    def deg_kernel(idx_hbm, ones_hbm, zero_hbm, out_hbm, idxb, ones, acc,
                   isem, ssem):
        c = lax.axis_index("c")
        s = lax.axis_index("s")
        pltpu.sync_copy(zero_hbm.at[pl.ds(s * rpt, rpt), :],
                        acc.at[pl.ds(s * rpt, rpt), :])
        pltpu.sync_copy(ones_hbm, ones)
        plsc.subcore_barrier()
        cbase = (c * _NT + s) * cpt

        def ic(j, q):
            return pltpu.make_async_copy(idx_hbm.at[cbase + j], idxb.at[q],
                                         isem)

        def sca(q):
            return pltpu.make_async_copy(ones, acc.at[idxb.at[q, 1]], ssem)

        ic(0, 0).start()
        ic(1, 1).start()

        def body(i, carry):
            q4 = lax.rem(i, 4)
            q4nn = lax.rem(i + 2, 4)
            ic(i, q4).wait()

            @pl.when(i >= 2)
            def _():
                sca(q4nn).wait()

            @pl.when(i + 2 < cpt)
            def _():
                ic(i + 2, q4nn).start()

            sca(q4).start(add=True)
            return carry

        lax.fori_loop(0, cpt, body, 0)
        sca(0).wait()
        sca(0).wait()
        plsc.subcore_barrier()
        pltpu.sync_copy(acc.at[pl.ds(s * rpt, rpt), :],
                        out_hbm.at[pl.ds(c * _NPAD + s * rpt, rpt), :])

    return deg_kernel


_RB = 2048


def _tc_a_body(deg_ref, x_ref, w_ref, hp_ref, dinv_ref, *, n):
    r = pl.program_id(0)
    deg = deg_ref[0, :, 0:1] + deg_ref[1, :, 0:1] + 1.0
    dinv = lax.rsqrt(jnp.maximum(deg, 1.0))
    rows = r * _RB + lax.broadcasted_iota(jnp.int32, (_RB, 1), 0)
    dinv = jnp.where(rows < n, dinv, 0.0)
    h = jnp.dot(x_ref[...], w_ref[...], preferred_element_type=jnp.float32)
    hp = h * dinv
    hp_ref[0] = hp[:, :_W]
    hp_ref[1] = hp[:, _W:]
    dinv_ref[...] = dinv


def _tc_a(deg2, x_p, w):
    grid = _NPAD // _RB
    return pl.pallas_call(
        functools.partial(_tc_a_body, n=10000),
        grid=(grid,),
        in_specs=[
            pl.BlockSpec((2, _RB, _W), lambda r: (0, r, 0)),
            pl.BlockSpec((_RB, x_p.shape[1]), lambda r: (r, 0)),
            pl.BlockSpec(w.shape, lambda r: (0, 0)),
        ],
        out_specs=[
            pl.BlockSpec((2, _RB, _W), lambda r: (0, r, 0)),
            pl.BlockSpec((_RB, 1), lambda r: (r, 0)),
        ],
        out_shape=[
            jax.ShapeDtypeStruct((2, _NPAD, _W), jnp.float32),
            jax.ShapeDtypeStruct((_NPAD, 1), jnp.float32),
        ],
    )(deg2, x_p, w)


def _tc_b_body(agg_ref, hp_ref, dinv_ref, b_ref, g_ref, be_ref, w_ref,
               out_ref, colsum, colsq, *, n, split_out):
    p = pl.program_id(0)
    r = pl.program_id(1)
    a = jnp.concatenate([agg_ref[0], agg_ref[1]], axis=1)
    hsl = jnp.concatenate([hp_ref[0], hp_ref[1]], axis=1)
    t = (a + hsl) * dinv_ref[...] + b_ref[...]

    @pl.when((p == 0) & (r == 0))
    def _():
        colsum[...] = jnp.zeros_like(colsum)
        colsq[...] = jnp.zeros_like(colsq)

    @pl.when(p == 0)
    def _():
        rows = r * _RB + lax.broadcasted_iota(jnp.int32, (_RB, 1), 0)
        tm = jnp.where(rows < n, t, 0.0)
        colsum[...] += jnp.sum(tm, axis=0, keepdims=True)
        colsq[...] += jnp.sum(tm * tm, axis=0, keepdims=True)

    @pl.when(p == 1)
    def _():
        mean = colsum[...] / n
        var = colsq[...] / n - mean * mean
        y = (t - mean) * lax.rsqrt(var + 1e-5) * g_ref[...] + be_ref[...]
        y = jnp.maximum(y, 0.0)
        hnew = jnp.dot(y, w_ref[...], preferred_element_type=jnp.float32)
        hp = hnew * dinv_ref[...]
        if split_out:
            out_ref[0] = hp[:, :_W]
            out_ref[1] = hp[:, _W:]
        else:
            out_ref[0] = hp
            out_ref[1] = jnp.zeros_like(hp)


def _tc_b(agg, hp_in, dinv, b, g, be, w, split_out):
    hcur = 2 * _W
    grid = _NPAD // _RB
    return pl.pallas_call(
        functools.partial(_tc_b_body, n=10000, split_out=split_out),
        grid=(2, grid),
        in_specs=[
            pl.BlockSpec((2, _RB, _W), lambda p, r: (0, r, 0)),
            pl.BlockSpec((2, _RB, _W), lambda p, r: (0, r, 0)),
            pl.BlockSpec((_RB, 1), lambda p, r: (r, 0)),
            pl.BlockSpec((1, hcur), lambda p, r: (0, 0)),
            pl.BlockSpec((1, hcur), lambda p, r: (0, 0)),
            pl.BlockSpec((1, hcur), lambda p, r: (0, 0)),
            pl.BlockSpec(w.shape, lambda p, r: (0, 0)),
        ],
        out_specs=pl.BlockSpec((2, _RB, _W), lambda p, r: (0, r * p, 0)),
        out_shape=jax.ShapeDtypeStruct((2, _NPAD, _W), jnp.float32),
        scratch_shapes=[
            pltpu.VMEM((1, hcur), jnp.float32),
            pltpu.VMEM((1, hcur), jnp.float32),
        ],
    )(agg, hp_in, dinv, b, g, be, w)


def _tc_c_body(agg_ref, hp_ref, dinv_ref, b_ref, out_ref, *, c):
    t = ((agg_ref[0] + agg_ref[1] + hp_ref[0]) * dinv_ref[...]
         + b_ref[...])
    col = lax.broadcasted_iota(jnp.int32, t.shape, 1)
    tm = jnp.where(col < c, t, -jnp.inf)
    mx = jnp.max(tm, axis=1, keepdims=True)
    e = jnp.exp(tm - mx)
    lse = jnp.log(jnp.sum(e, axis=1, keepdims=True)) + mx
    out_ref[...] = t - lse


def _tc_c(agg, hp_in, dinv, b_p, c):
    grid = _NPAD // _RB
    return pl.pallas_call(
        functools.partial(_tc_c_body, c=c),
        grid=(grid,),
        in_specs=[
            pl.BlockSpec((2, _RB, _W), lambda r: (0, r, 0)),
            pl.BlockSpec((2, _RB, _W), lambda r: (0, r, 0)),
            pl.BlockSpec((_RB, 1), lambda r: (r, 0)),
            pl.BlockSpec((1, _W), lambda r: (0, 0)),
        ],
        out_specs=pl.BlockSpec((_RB, _W), lambda r: (r, 0)),
        out_shape=jax.ShapeDtypeStruct((_NPAD, _W), jnp.float32),
    )(agg, hp_in, dinv, b_p)


def kernel(x, edge_index, W0, b0, g0, be0, W1, b1, g1, be1, W2, b2):
    n, d = x.shape
    h = W0.shape[1]
    c = W2.shape[1]

    pad_e = _EPAD - edge_index.shape[1]
    src_p = jnp.concatenate(
        [edge_index[0], jnp.full((pad_e,), n, jnp.int32)])
    dst_p = jnp.concatenate(
        [edge_index[1], jnp.full((pad_e,), n, jnp.int32)])
    nch = _EPAD // _K
    idx_es = jnp.stack(
        [src_p.reshape(nch, _K), dst_p.reshape(nch, _K)], axis=1)
    idx_fs = jnp.concatenate(
        [idx_es, idx_es + jnp.array([_NPAD, 0], jnp.int32)[None, :, None]])
    x_p = jnp.pad(x, ((0, _NPAD - n), (0, 0)))
    w2_p = jnp.pad(W2, ((0, 0), (0, _W - c)))
    b2_p = jnp.pad(b2, ((0, _W - c),)).reshape(1, _W)
    z128 = jnp.zeros((_NPAD, _W), jnp.float32)

    agg_fs = _make_agg(edge_split=False)
    agg_es = _make_agg(edge_split=True)

    deg2 = _make_deg()(idx_es, jnp.ones((_K, _W), jnp.float32),
                       z128).reshape(2, _NPAD, _W)
    hp0, dinv = _tc_a(deg2, x_p, W0)

    a0 = agg_fs(hp0.reshape(2 * _NPAD, _W), z128, idx_fs).reshape(
        2, _NPAD, _W)
    hp1 = _tc_b(a0, hp0, dinv, b0.reshape(1, h), g0.reshape(1, h),
                be0.reshape(1, h), W1, split_out=True)
    a1 = agg_fs(hp1.reshape(2 * _NPAD, _W), z128, idx_fs).reshape(
        2, _NPAD, _W)
    hp2 = _tc_b(a1, hp1, dinv, b1.reshape(1, h), g1.reshape(1, h),
                be1.reshape(1, h), w2_p, split_out=False)

    a2 = agg_es(hp2.reshape(2 * _NPAD, _W), z128, idx_es).reshape(
        2, _NPAD, _W)
    out = _tc_c(a2, hp2, dinv, b2_p, c)
    return out[:n, :c]
